# baseline (speedup 1.0000x reference)
import math

import jax
import jax.numpy as jnp
from jax.experimental import pallas as pl
from jax.experimental.pallas import tpu as pltpu

_C = 100000
_SMOOTHING = 0.1
_CONF = 1.0 - _SMOOTHING
_EPS = _SMOOTHING / (_C - 1)

_R = 8
_K = 8192


def _loss_kernel(tgt_s_ref, tgt_v_ref, pred_ref, out_ref, m_ref, s_ref,
                 sp_ref, ts_ref):
    rb = pl.program_id(0)
    kc = pl.program_id(1)
    nk = pl.num_programs(1)

    x = pred_ref[...]
    base = kc * _K
    is_tail = kc == nk - 1

    @pl.when(kc == 0)
    def _init_ts():
        ts_ref[...] = jnp.zeros((_R, 1), jnp.float32)

    hit = None
    for i in range(_R):
        t = tgt_s_ref[0, 0, i]
        h = jnp.logical_and(t >= base, t < base + _K)
        hit = h if hit is None else jnp.logical_or(hit, h)

    @pl.when(hit)
    def _tsel():
        col = jax.lax.broadcasted_iota(jnp.int32, (_R, _K), 1) + base
        tgtv = tgt_v_ref[0, 0, :].reshape(_R, 1)
        ts_ref[...] = ts_ref[...] + jnp.sum(
            jnp.where(col == tgtv, x, 0.0), axis=1, keepdims=True)

    @pl.when(jnp.logical_not(is_tail))
    def _dense():
        cmax = jnp.max(x, axis=1, keepdims=True)

        @pl.when(kc == 0)
        def _first():
            m_ref[...] = cmax
            s_ref[...] = jnp.sum(jnp.exp(x - cmax), axis=1, keepdims=True)
            sp_ref[...] = jnp.sum(x, axis=1, keepdims=True)

        @pl.when(kc > 0)
        def _update():
            m_old = m_ref[...]
            m_new = jnp.maximum(m_old, cmax)
            s_ref[...] = s_ref[...] * jnp.exp(m_old - m_new) + jnp.sum(
                jnp.exp(x - m_new), axis=1, keepdims=True)
            m_ref[...] = m_new
            sp_ref[...] = sp_ref[...] + jnp.sum(x, axis=1, keepdims=True)

    @pl.when(is_tail)
    def _tail():
        col = jax.lax.broadcasted_iota(jnp.int32, (_R, _K), 1) + base
        valid = col < _C
        xm = jnp.where(valid, x, -jnp.inf)
        cmax = jnp.max(xm, axis=1, keepdims=True)
        m_old = m_ref[...]
        m_new = jnp.maximum(m_old, cmax)
        s_ref[...] = s_ref[...] * jnp.exp(m_old - m_new) + jnp.sum(
            jnp.exp(xm - m_new), axis=1, keepdims=True)
        m_ref[...] = m_new
        sp_ref[...] = sp_ref[...] + jnp.sum(
            jnp.where(valid, x, 0.0), axis=1, keepdims=True)

        lse = m_new + jnp.log(s_ref[...])
        rowsum_logp = sp_ref[...] - _C * lse
        logp_t = ts_ref[...] - lse
        contrib = -(_EPS * rowsum_logp + (_CONF - _EPS) * logp_t)
        val = jnp.sum(contrib)

        @pl.when(rb == 0)
        def _():
            out_ref[0, 0] = val

        @pl.when(rb > 0)
        def _():
            out_ref[0, 0] = out_ref[0, 0] + val


@jax.jit
def kernel(pred, target):
    B = pred.shape[0]
    nb = B // _R
    nk = pl.cdiv(_C, _K)
    tgt3 = target.astype(jnp.int32).reshape(nb, 1, _R)

    acc = pl.pallas_call(
        _loss_kernel,
        grid=(nb, nk),
        in_specs=[
            pl.BlockSpec((1, 1, _R), lambda rb, kc: (rb, 0, 0),
                         memory_space=pltpu.SMEM),
            pl.BlockSpec((1, 1, _R), lambda rb, kc: (rb, 0, 0)),
            pl.BlockSpec((_R, _K), lambda rb, kc: (rb, kc)),
        ],
        out_specs=pl.BlockSpec(
            (1, 1), lambda rb, kc: (0, 0), memory_space=pltpu.SMEM),
        out_shape=jax.ShapeDtypeStruct((1, 1), jnp.float32),
        scratch_shapes=[
            pltpu.VMEM((_R, 1), jnp.float32),
            pltpu.VMEM((_R, 1), jnp.float32),
            pltpu.VMEM((_R, 1), jnp.float32),
            pltpu.VMEM((_R, 1), jnp.float32),
        ],
    )(tgt3, tgt3, pred)

    k0 = (_C - 1) * _EPS * math.log(_EPS) + _CONF * math.log(_CONF)
    return (acc[0, 0] + B * k0) / (B * _C)


# speedup vs baseline: 1.0165x; 1.0165x over previous
import math

import jax
import jax.numpy as jnp
from jax.experimental import pallas as pl
from jax.experimental.pallas import tpu as pltpu

_C = 100000
_SMOOTHING = 0.1
_CONF = 1.0 - _SMOOTHING
_EPS = _SMOOTHING / (_C - 1)

_R = 8
_K = 8192
_P = 8
_W = _K // _P
_NEG_INF = float("-inf")


def _tree(vals, op):
    while len(vals) > 1:
        nxt = [op(vals[i], vals[i + 1]) for i in range(0, len(vals) - 1, 2)]
        if len(vals) % 2:
            nxt.append(vals[-1])
        vals = nxt
    return vals[0]


def _slice(pred_ref, i):
    return pred_ref[:, pl.ds(i * _W, _W)]


def _mask(i, base):
    col = jax.lax.broadcasted_iota(jnp.int32, (_R, _W), 1) + (base + i * _W)
    return col < _C


def _chunk_max(pred_ref, base, masked):
    parts = []
    for i in range(_P):
        xi = _slice(pred_ref, i)
        if masked:
            xi = jnp.where(_mask(i, base), xi, _NEG_INF)
        parts.append(jnp.max(xi, axis=1, keepdims=True))
    return _tree(parts, jnp.maximum)


def _chunk_stats(pred_ref, base, masked, m_new):
    se_parts = []
    sp_parts = []
    for i in range(_P):
        xi = _slice(pred_ref, i)
        if masked:
            v = _mask(i, base)
            e = jnp.exp(jnp.where(v, xi, _NEG_INF) - m_new)
            xs = jnp.where(v, xi, 0.0)
        else:
            e = jnp.exp(xi - m_new)
            xs = xi
        se_parts.append(jnp.sum(e, axis=1, keepdims=True))
        sp_parts.append(jnp.sum(xs, axis=1, keepdims=True))
    return _tree(se_parts, jnp.add), _tree(sp_parts, jnp.add)


def _loss_kernel(tgt_s_ref, tgt_v_ref, pred_ref, out_ref, m_ref, s_ref,
                 sp_ref, ts_ref):
    rb = pl.program_id(0)
    kc = pl.program_id(1)
    nk = pl.num_programs(1)

    base = kc * _K
    is_tail = kc == nk - 1

    @pl.when(kc == 0)
    def _init():
        m_ref[...] = jnp.full((_R, 1), _NEG_INF, jnp.float32)
        s_ref[...] = jnp.zeros((_R, 1), jnp.float32)
        sp_ref[...] = jnp.zeros((_R, 1), jnp.float32)
        ts_ref[...] = jnp.zeros((_R, 1), jnp.float32)

    hit = None
    for i in range(_R):
        t = tgt_s_ref[0, 0, i]
        h = jnp.logical_and(t >= base, t < base + _K)
        hit = h if hit is None else jnp.logical_or(hit, h)

    @pl.when(hit)
    def _tsel():
        tgtv = tgt_v_ref[0, 0, :].reshape(_R, 1)
        parts = []
        for i in range(_P):
            xi = _slice(pred_ref, i)
            col = jax.lax.broadcasted_iota(
                jnp.int32, (_R, _W), 1) + (base + i * _W)
            parts.append(jnp.sum(
                jnp.where(col == tgtv, xi, 0.0), axis=1, keepdims=True))
        ts_ref[...] = ts_ref[...] + _tree(parts, jnp.add)

    def _update(masked):
        cmax = _chunk_max(pred_ref, base, masked)
        m_old = m_ref[...]
        m_new = jnp.maximum(m_old, cmax)
        se, sp = _chunk_stats(pred_ref, base, masked, m_new)
        s_new = s_ref[...] * jnp.exp(m_old - m_new) + se
        s_ref[...] = s_new
        m_ref[...] = m_new
        sp_new = sp_ref[...] + sp
        sp_ref[...] = sp_new
        return m_new, s_new, sp_new

    @pl.when(jnp.logical_not(is_tail))
    def _dense():
        _update(masked=False)

    @pl.when(is_tail)
    def _tail():
        m_new, s_new, sp_new = _update(masked=True)

        lse = m_new + jnp.log(s_new)
        rowsum_logp = sp_new - _C * lse
        logp_t = ts_ref[...] - lse
        contrib = -(_EPS * rowsum_logp + (_CONF - _EPS) * logp_t)
        val = jnp.sum(contrib)

        @pl.when(rb == 0)
        def _():
            out_ref[0, 0] = val

        @pl.when(rb > 0)
        def _():
            out_ref[0, 0] = out_ref[0, 0] + val


@jax.jit
def kernel(pred, target):
    B = pred.shape[0]
    nb = B // _R
    nk = pl.cdiv(_C, _K)
    tgt3 = target.astype(jnp.int32).reshape(nb, 1, _R)

    acc = pl.pallas_call(
        _loss_kernel,
        grid=(nb, nk),
        in_specs=[
            pl.BlockSpec((1, 1, _R), lambda rb, kc: (rb, 0, 0),
                         memory_space=pltpu.SMEM),
            pl.BlockSpec((1, 1, _R), lambda rb, kc: (rb, 0, 0)),
            pl.BlockSpec((_R, _K), lambda rb, kc: (rb, kc)),
        ],
        out_specs=pl.BlockSpec(
            (1, 1), lambda rb, kc: (0, 0), memory_space=pltpu.SMEM),
        out_shape=jax.ShapeDtypeStruct((1, 1), jnp.float32),
        scratch_shapes=[
            pltpu.VMEM((_R, 1), jnp.float32),
            pltpu.VMEM((_R, 1), jnp.float32),
            pltpu.VMEM((_R, 1), jnp.float32),
            pltpu.VMEM((_R, 1), jnp.float32),
        ],
    )(tgt3, tgt3, pred)

    k0 = (_C - 1) * _EPS * math.log(_EPS) + _CONF * math.log(_CONF)
    return (acc[0, 0] + B * k0) / (B * _C)


# speedup vs baseline: 1.1882x; 1.1690x over previous
import math

import jax
import jax.numpy as jnp
from jax.experimental import pallas as pl
from jax.experimental.pallas import tpu as pltpu

_C = 100000
_SMOOTHING = 0.1
_CONF = 1.0 - _SMOOTHING
_EPS = _SMOOTHING / (_C - 1)

_R = 8
_K = 8192
_L = 128
_T = _K // _L
_NK = -(-_C // _K)
_TAIL_BASE = (_NK - 1) * _K
_TAIL_TILES = -(-(_C - _TAIL_BASE) // _L)
_TAIL_REM = _C - _TAIL_BASE - (_TAIL_TILES - 1) * _L
_G = 8
_NEG_INF = float("-inf")


def _tree(vals, op):
    while len(vals) > 1:
        nxt = [op(vals[i], vals[i + 1]) for i in range(0, len(vals) - 1, 2)]
        if len(vals) % 2:
            nxt.append(vals[-1])
        vals = nxt
    return vals[0]


def _tile(pred_ref, t):
    return pred_ref[:, pl.ds(t * _L, _L)]


def _lane_iota():
    return jax.lax.broadcasted_iota(jnp.int32, (_R, _L), 1)


def _sweep_max(pred_ref, ntiles, mask_last):
    accs = []
    for g0 in range(0, ntiles, _G):
        acc = None
        for t in range(g0, min(g0 + _G, ntiles)):
            x = _tile(pred_ref, t)
            if mask_last and t == ntiles - 1:
                x = jnp.where(_lane_iota() < _TAIL_REM, x, _NEG_INF)
            acc = x if acc is None else jnp.maximum(acc, x)
        accs.append(acc)
    return _tree(accs, jnp.maximum)


def _sweep_stats(pred_ref, ntiles, mask_last, m_new):
    se_accs = []
    sp_accs = []
    for g0 in range(0, ntiles, _G):
        se = None
        sp = None
        for t in range(g0, min(g0 + _G, ntiles)):
            x = _tile(pred_ref, t)
            if mask_last and t == ntiles - 1:
                e = jnp.exp(jnp.where(_lane_iota() < _TAIL_REM, x,
                                      _NEG_INF) - m_new)
                xs = jnp.where(_lane_iota() < _TAIL_REM, x, 0.0)
            else:
                e = jnp.exp(x - m_new)
                xs = x
            se = e if se is None else se + e
            sp = xs if sp is None else sp + xs
        se_accs.append(se)
        sp_accs.append(sp)
    return _tree(se_accs, jnp.add), _tree(sp_accs, jnp.add)


def _loss_kernel(tgt_s_ref, tgt_v_ref, pred_ref, out_ref, m_ref, s_ref,
                 sp_ref, ts_ref):
    rb = pl.program_id(0)
    kc = pl.program_id(1)

    base = kc * _K
    is_tail = kc == _NK - 1

    @pl.when(kc == 0)
    def _init():
        m_ref[...] = jnp.full((_R, _L), _NEG_INF, jnp.float32)
        s_ref[...] = jnp.zeros((_R, _L), jnp.float32)
        sp_ref[...] = jnp.zeros((_R, _L), jnp.float32)
        ts_ref[...] = jnp.zeros((_R, _L), jnp.float32)

    hit = None
    for i in range(_R):
        t = tgt_s_ref[0, 0, i]
        h = jnp.logical_and(t >= base, t < base + _K)
        hit = h if hit is None else jnp.logical_or(hit, h)

    @pl.when(hit)
    def _tsel():
        tgtv = jnp.broadcast_to(tgt_v_ref[0, 0, :].reshape(_R, 1), (_R, _L))
        li = _lane_iota()
        accs = []
        for g0 in range(0, _T, _G):
            acc = None
            for t in range(g0, min(g0 + _G, _T)):
                col = li + (base + t * _L)
                v = jnp.where(col == tgtv, _tile(pred_ref, t), 0.0)
                acc = v if acc is None else acc + v
            accs.append(acc)
        ts_ref[...] = ts_ref[...] + _tree(accs, jnp.add)

    def _update(ntiles, mask_last):
        cmax = _sweep_max(pred_ref, ntiles, mask_last)
        m_old = m_ref[...]
        m_new = jnp.maximum(m_old, cmax)
        se, sp = _sweep_stats(pred_ref, ntiles, mask_last, m_new)
        s_new = s_ref[...] * jnp.exp(m_old - m_new) + se
        s_ref[...] = s_new
        m_ref[...] = m_new
        sp_new = sp_ref[...] + sp
        sp_ref[...] = sp_new
        return m_new, s_new, sp_new

    @pl.when(jnp.logical_not(is_tail))
    def _dense():
        _update(_T, False)

    @pl.when(is_tail)
    def _tail():
        m, s, sp = _update(_TAIL_TILES, True)

        mx = jnp.max(m, axis=1, keepdims=True)
        sx = jnp.sum(s * jnp.exp(m - mx), axis=1, keepdims=True)
        spx = jnp.sum(sp, axis=1, keepdims=True)
        tsx = jnp.sum(ts_ref[...], axis=1, keepdims=True)
        lse = mx + jnp.log(sx)
        rowsum_logp = spx - _C * lse
        logp_t = tsx - lse
        contrib = -(_EPS * rowsum_logp + (_CONF - _EPS) * logp_t)
        val = jnp.sum(contrib)

        @pl.when(rb == 0)
        def _():
            out_ref[0, 0] = val

        @pl.when(rb > 0)
        def _():
            out_ref[0, 0] = out_ref[0, 0] + val


@jax.jit
def kernel(pred, target):
    B = pred.shape[0]
    nb = B // _R
    tgt3 = target.astype(jnp.int32).reshape(nb, 1, _R)

    acc = pl.pallas_call(
        _loss_kernel,
        grid=(nb, _NK),
        in_specs=[
            pl.BlockSpec((1, 1, _R), lambda rb, kc: (rb, 0, 0),
                         memory_space=pltpu.SMEM),
            pl.BlockSpec((1, 1, _R), lambda rb, kc: (rb, 0, 0)),
            pl.BlockSpec((_R, _K), lambda rb, kc: (rb, kc)),
        ],
        out_specs=pl.BlockSpec(
            (1, 1), lambda rb, kc: (0, 0), memory_space=pltpu.SMEM),
        out_shape=jax.ShapeDtypeStruct((1, 1), jnp.float32),
        scratch_shapes=[
            pltpu.VMEM((_R, _L), jnp.float32),
            pltpu.VMEM((_R, _L), jnp.float32),
            pltpu.VMEM((_R, _L), jnp.float32),
            pltpu.VMEM((_R, _L), jnp.float32),
        ],
    )(tgt3, tgt3, pred)

    k0 = (_C - 1) * _EPS * math.log(_EPS) + _CONF * math.log(_CONF)
    return (acc[0, 0] + B * k0) / (B * _C)


# speedup vs baseline: 1.6549x; 1.3928x over previous
import math

import jax
import jax.numpy as jnp
from jax.experimental import pallas as pl
from jax.experimental.pallas import tpu as pltpu

_C = 100000
_SMOOTHING = 0.1
_CONF = 1.0 - _SMOOTHING
_EPS = _SMOOTHING / (_C - 1)

_R = 8
_K = 16384
_L = 128
_T = _K // _L
_NK = -(-_C // _K)
_TAIL_BASE = (_NK - 1) * _K
_TAIL_TILES = -(-(_C - _TAIL_BASE) // _L)
_TAIL_REM = _C - _TAIL_BASE - (_TAIL_TILES - 1) * _L
_G = 8
_NEG_INF = float("-inf")


def _tree(vals, op):
    while len(vals) > 1:
        nxt = [op(vals[i], vals[i + 1]) for i in range(0, len(vals) - 1, 2)]
        if len(vals) % 2:
            nxt.append(vals[-1])
        vals = nxt
    return vals[0]


def _tile(pred_ref, t):
    return pred_ref[:, pl.ds(t * _L, _L)]


def _lane_iota():
    return jax.lax.broadcasted_iota(jnp.int32, (_R, _L), 1)


def _sweep_max(pred_ref, ntiles, mask_last):
    accs = []
    for g0 in range(0, ntiles, _G):
        acc = None
        for t in range(g0, min(g0 + _G, ntiles)):
            x = _tile(pred_ref, t)
            if mask_last and t == ntiles - 1:
                x = jnp.where(_lane_iota() < _TAIL_REM, x, _NEG_INF)
            acc = x if acc is None else jnp.maximum(acc, x)
        accs.append(acc)
    return _tree(accs, jnp.maximum)


def _sweep_stats(pred_ref, ntiles, mask_last, m_new):
    se_accs = []
    sp_accs = []
    for g0 in range(0, ntiles, _G):
        se = None
        sp = None
        for t in range(g0, min(g0 + _G, ntiles)):
            x = _tile(pred_ref, t)
            if mask_last and t == ntiles - 1:
                e = jnp.exp(jnp.where(_lane_iota() < _TAIL_REM, x,
                                      _NEG_INF) - m_new)
                xs = jnp.where(_lane_iota() < _TAIL_REM, x, 0.0)
            else:
                e = jnp.exp(x - m_new)
                xs = x
            se = e if se is None else se + e
            sp = xs if sp is None else sp + xs
        se_accs.append(se)
        sp_accs.append(sp)
    return _tree(se_accs, jnp.add), _tree(sp_accs, jnp.add)


def _loss_kernel(tgt_s_ref, tgt_v_ref, pred_ref, out_ref, m_ref, s_ref,
                 sp_ref, ts_ref):
    rb = pl.program_id(0)
    kc = pl.program_id(1)

    base = kc * _K
    is_tail = kc == _NK - 1

    @pl.when(kc == 0)
    def _init():
        m_ref[...] = jnp.full((_R, _L), _NEG_INF, jnp.float32)
        s_ref[...] = jnp.zeros((_R, _L), jnp.float32)
        sp_ref[...] = jnp.zeros((_R, _L), jnp.float32)
        ts_ref[...] = jnp.zeros((_R, _L), jnp.float32)

    hit = None
    for i in range(_R):
        t = tgt_s_ref[0, 0, i]
        h = jnp.logical_and(t >= base, t < base + _K)
        hit = h if hit is None else jnp.logical_or(hit, h)

    @pl.when(hit)
    def _tsel():
        tgtv = jnp.broadcast_to(tgt_v_ref[0, 0, :].reshape(_R, 1), (_R, _L))
        li = _lane_iota()
        accs = []
        for g0 in range(0, _T, _G):
            acc = None
            for t in range(g0, min(g0 + _G, _T)):
                col = li + (base + t * _L)
                v = jnp.where(col == tgtv, _tile(pred_ref, t), 0.0)
                acc = v if acc is None else acc + v
            accs.append(acc)
        ts_ref[...] = ts_ref[...] + _tree(accs, jnp.add)

    def _update(ntiles, mask_last):
        cmax = _sweep_max(pred_ref, ntiles, mask_last)
        m_old = m_ref[...]
        m_new = jnp.maximum(m_old, cmax)
        se, sp = _sweep_stats(pred_ref, ntiles, mask_last, m_new)
        s_new = s_ref[...] * jnp.exp(m_old - m_new) + se
        s_ref[...] = s_new
        m_ref[...] = m_new
        sp_new = sp_ref[...] + sp
        sp_ref[...] = sp_new
        return m_new, s_new, sp_new

    @pl.when(jnp.logical_not(is_tail))
    def _dense():
        _update(_T, False)

    @pl.when(is_tail)
    def _tail():
        m, s, sp = _update(_TAIL_TILES, True)

        mx = jnp.max(m, axis=1, keepdims=True)
        sx = jnp.sum(s * jnp.exp(m - mx), axis=1, keepdims=True)
        spx = jnp.sum(sp, axis=1, keepdims=True)
        tsx = jnp.sum(ts_ref[...], axis=1, keepdims=True)
        lse = mx + jnp.log(sx)
        rowsum_logp = spx - _C * lse
        logp_t = tsx - lse
        contrib = -(_EPS * rowsum_logp + (_CONF - _EPS) * logp_t)
        val = jnp.sum(contrib)

        @pl.when(rb == 0)
        def _():
            out_ref[0, 0] = val

        @pl.when(rb > 0)
        def _():
            out_ref[0, 0] = out_ref[0, 0] + val


@jax.jit
def kernel(pred, target):
    B = pred.shape[0]
    nb = B // _R
    tgt3 = target.astype(jnp.int32).reshape(nb, 1, _R)

    acc = pl.pallas_call(
        _loss_kernel,
        grid=(nb, _NK),
        in_specs=[
            pl.BlockSpec((1, 1, _R), lambda rb, kc: (rb, 0, 0),
                         memory_space=pltpu.SMEM),
            pl.BlockSpec((1, 1, _R), lambda rb, kc: (rb, 0, 0)),
            pl.BlockSpec((_R, _K), lambda rb, kc: (rb, kc)),
        ],
        out_specs=pl.BlockSpec(
            (1, 1), lambda rb, kc: (0, 0), memory_space=pltpu.SMEM),
        out_shape=jax.ShapeDtypeStruct((1, 1), jnp.float32),
        scratch_shapes=[
            pltpu.VMEM((_R, _L), jnp.float32),
            pltpu.VMEM((_R, _L), jnp.float32),
            pltpu.VMEM((_R, _L), jnp.float32),
            pltpu.VMEM((_R, _L), jnp.float32),
        ],
    )(tgt3, tgt3, pred)

    k0 = (_C - 1) * _EPS * math.log(_EPS) + _CONF * math.log(_CONF)
    return (acc[0, 0] + B * k0) / (B * _C)


# speedup vs baseline: 1.9741x; 1.1929x over previous
import math

import jax
import jax.numpy as jnp
from jax.experimental import pallas as pl
from jax.experimental.pallas import tpu as pltpu

_C = 100000
_SMOOTHING = 0.1
_CONF = 1.0 - _SMOOTHING
_EPS = _SMOOTHING / (_C - 1)

_R = 8
_K = 32768
_L = 128
_T = _K // _L
_NK = -(-_C // _K)
_TAIL_BASE = (_NK - 1) * _K
_TAIL_TILES = -(-(_C - _TAIL_BASE) // _L)
_TAIL_REM = _C - _TAIL_BASE - (_TAIL_TILES - 1) * _L
_G = 8
_NEG_INF = float("-inf")


def _tree(vals, op):
    while len(vals) > 1:
        nxt = [op(vals[i], vals[i + 1]) for i in range(0, len(vals) - 1, 2)]
        if len(vals) % 2:
            nxt.append(vals[-1])
        vals = nxt
    return vals[0]


def _tile(pred_ref, t):
    return pred_ref[:, pl.ds(t * _L, _L)]


def _lane_iota():
    return jax.lax.broadcasted_iota(jnp.int32, (_R, _L), 1)


def _sweep_max(pred_ref, ntiles, mask_last):
    accs = []
    for g0 in range(0, ntiles, _G):
        acc = None
        for t in range(g0, min(g0 + _G, ntiles)):
            x = _tile(pred_ref, t)
            if mask_last and t == ntiles - 1:
                x = jnp.where(_lane_iota() < _TAIL_REM, x, _NEG_INF)
            acc = x if acc is None else jnp.maximum(acc, x)
        accs.append(acc)
    return _tree(accs, jnp.maximum)


def _sweep_stats(pred_ref, ntiles, mask_last, m_new):
    se_accs = []
    sp_accs = []
    for g0 in range(0, ntiles, _G):
        se = None
        sp = None
        for t in range(g0, min(g0 + _G, ntiles)):
            x = _tile(pred_ref, t)
            if mask_last and t == ntiles - 1:
                e = jnp.exp(jnp.where(_lane_iota() < _TAIL_REM, x,
                                      _NEG_INF) - m_new)
                xs = jnp.where(_lane_iota() < _TAIL_REM, x, 0.0)
            else:
                e = jnp.exp(x - m_new)
                xs = x
            se = e if se is None else se + e
            sp = xs if sp is None else sp + xs
        se_accs.append(se)
        sp_accs.append(sp)
    return _tree(se_accs, jnp.add), _tree(sp_accs, jnp.add)


def _loss_kernel(tgt_s_ref, tgt_v_ref, pred_ref, out_ref, m_ref, s_ref,
                 sp_ref, ts_ref):
    rb = pl.program_id(0)
    kc = pl.program_id(1)

    base = kc * _K
    is_tail = kc == _NK - 1

    @pl.when(kc == 0)
    def _init():
        m_ref[...] = jnp.full((_R, _L), _NEG_INF, jnp.float32)
        s_ref[...] = jnp.zeros((_R, _L), jnp.float32)
        sp_ref[...] = jnp.zeros((_R, _L), jnp.float32)
        ts_ref[...] = jnp.zeros((_R, _L), jnp.float32)

    hit = None
    for i in range(_R):
        t = tgt_s_ref[0, 0, i]
        h = jnp.logical_and(t >= base, t < base + _K)
        hit = h if hit is None else jnp.logical_or(hit, h)

    @pl.when(hit)
    def _tsel():
        tgtv = jnp.broadcast_to(tgt_v_ref[0, 0, :].reshape(_R, 1), (_R, _L))
        li = _lane_iota()
        accs = []
        for g0 in range(0, _T, _G):
            acc = None
            for t in range(g0, min(g0 + _G, _T)):
                col = li + (base + t * _L)
                v = jnp.where(col == tgtv, _tile(pred_ref, t), 0.0)
                acc = v if acc is None else acc + v
            accs.append(acc)
        ts_ref[...] = ts_ref[...] + _tree(accs, jnp.add)

    def _update(ntiles, mask_last):
        cmax = _sweep_max(pred_ref, ntiles, mask_last)
        m_old = m_ref[...]
        m_new = jnp.maximum(m_old, cmax)
        se, sp = _sweep_stats(pred_ref, ntiles, mask_last, m_new)
        s_new = s_ref[...] * jnp.exp(m_old - m_new) + se
        s_ref[...] = s_new
        m_ref[...] = m_new
        sp_new = sp_ref[...] + sp
        sp_ref[...] = sp_new
        return m_new, s_new, sp_new

    @pl.when(jnp.logical_not(is_tail))
    def _dense():
        _update(_T, False)

    @pl.when(is_tail)
    def _tail():
        m, s, sp = _update(_TAIL_TILES, True)

        mx = jnp.max(m, axis=1, keepdims=True)
        sx = jnp.sum(s * jnp.exp(m - mx), axis=1, keepdims=True)
        spx = jnp.sum(sp, axis=1, keepdims=True)
        tsx = jnp.sum(ts_ref[...], axis=1, keepdims=True)
        lse = mx + jnp.log(sx)
        rowsum_logp = spx - _C * lse
        logp_t = tsx - lse
        contrib = -(_EPS * rowsum_logp + (_CONF - _EPS) * logp_t)
        val = jnp.sum(contrib)

        @pl.when(rb == 0)
        def _():
            out_ref[0, 0] = val

        @pl.when(rb > 0)
        def _():
            out_ref[0, 0] = out_ref[0, 0] + val


@jax.jit
def kernel(pred, target):
    B = pred.shape[0]
    nb = B // _R
    tgt3 = target.astype(jnp.int32).reshape(nb, 1, _R)

    acc = pl.pallas_call(
        _loss_kernel,
        grid=(nb, _NK),
        in_specs=[
            pl.BlockSpec((1, 1, _R), lambda rb, kc: (rb, 0, 0),
                         memory_space=pltpu.SMEM),
            pl.BlockSpec((1, 1, _R), lambda rb, kc: (rb, 0, 0)),
            pl.BlockSpec((_R, _K), lambda rb, kc: (rb, kc)),
        ],
        out_specs=pl.BlockSpec(
            (1, 1), lambda rb, kc: (0, 0), memory_space=pltpu.SMEM),
        out_shape=jax.ShapeDtypeStruct((1, 1), jnp.float32),
        scratch_shapes=[
            pltpu.VMEM((_R, _L), jnp.float32),
            pltpu.VMEM((_R, _L), jnp.float32),
            pltpu.VMEM((_R, _L), jnp.float32),
            pltpu.VMEM((_R, _L), jnp.float32),
        ],
    )(tgt3, tgt3, pred)

    k0 = (_C - 1) * _EPS * math.log(_EPS) + _CONF * math.log(_CONF)
    return (acc[0, 0] + B * k0) / (B * _C)


# speedup vs baseline: 2.7992x; 1.4180x over previous
import math

import jax
import jax.numpy as jnp
from jax.experimental import pallas as pl
from jax.experimental.pallas import tpu as pltpu

_C = 100000
_SMOOTHING = 0.1
_CONF = 1.0 - _SMOOTHING
_EPS = _SMOOTHING / (_C - 1)

_R = 8
_L = 128
_NT = -(-_C // _L)
_CPAD = _NT * _L
_TAIL_REM = _C - (_NT - 1) * _L
_NCH = 16
_SEC = 8192
_NSEC = -(-_C // _SEC)
_NEG_INF = float("-inf")


def _tree(vals, op):
    while len(vals) > 1:
        nxt = [op(vals[i], vals[i + 1]) for i in range(0, len(vals) - 1, 2)]
        if len(vals) % 2:
            nxt.append(vals[-1])
        vals = nxt
    return vals[0]


def _tile(pred_ref, t):
    return pred_ref[:, pl.ds(t * _L, _L)]


def _lane_iota():
    return jax.lax.broadcasted_iota(jnp.int32, (_R, _L), 1)


def _chains(n):
    per = -(-n // _NCH)
    for c in range(_NCH):
        lo = c * per
        hi = min(lo + per, n)
        if lo < hi:
            yield range(lo, hi)


def _masked(x, t, fill):
    if t == _NT - 1:
        return jnp.where(_lane_iota() < _TAIL_REM, x, fill)
    return x


def _sweep_max(pred_ref):
    accs = []
    for chain in _chains(_NT):
        acc = None
        for t in chain:
            x = _masked(_tile(pred_ref, t), t, _NEG_INF)
            acc = x if acc is None else jnp.maximum(acc, x)
        accs.append(acc)
    return _tree(accs, jnp.maximum)


def _sweep_stats(pred_ref, m):
    se_accs = []
    sp_accs = []
    for chain in _chains(_NT):
        se = None
        sp = None
        for t in chain:
            x = _tile(pred_ref, t)
            e = jnp.exp(_masked(x, t, _NEG_INF) - m)
            xs = _masked(x, t, 0.0)
            se = e if se is None else se + e
            sp = xs if sp is None else sp + xs
        se_accs.append(se)
        sp_accs.append(sp)
    return _tree(se_accs, jnp.add), _tree(sp_accs, jnp.add)


def _loss_kernel(tgt_s_ref, tgt_v_ref, pred_ref, out_ref, ts_ref):
    rb = pl.program_id(0)

    ts_ref[...] = jnp.zeros((_R, _L), jnp.float32)

    tgtv = jnp.broadcast_to(tgt_v_ref[0, 0, :].reshape(_R, 1), (_R, _L))
    li = _lane_iota()
    for sec in range(_NSEC):
        lo = sec * _SEC
        hi = min(lo + _SEC, _C)
        hit = None
        for i in range(_R):
            t = tgt_s_ref[0, 0, i]
            h = jnp.logical_and(t >= lo, t < hi)
            hit = h if hit is None else jnp.logical_or(hit, h)

        @pl.when(hit)
        def _tsel(lo=lo, hi=hi):
            accs = []
            t0 = lo // _L
            t1 = -(-hi // _L)
            for g0 in range(t0, t1, 8):
                acc = None
                for t in range(g0, min(g0 + 8, t1)):
                    col = li + t * _L
                    v = jnp.where(col == tgtv, _tile(pred_ref, t), 0.0)
                    acc = v if acc is None else acc + v
                accs.append(acc)
            ts_ref[...] = ts_ref[...] + _tree(accs, jnp.add)

    m = _sweep_max(pred_ref)
    se, sp = _sweep_stats(pred_ref, m)

    mx = jnp.max(m, axis=1, keepdims=True)
    sx = jnp.sum(se * jnp.exp(m - mx), axis=1, keepdims=True)
    spx = jnp.sum(sp, axis=1, keepdims=True)
    tsx = jnp.sum(ts_ref[...], axis=1, keepdims=True)
    lse = mx + jnp.log(sx)
    rowsum_logp = spx - _C * lse
    logp_t = tsx - lse
    contrib = -(_EPS * rowsum_logp + (_CONF - _EPS) * logp_t)
    val = jnp.sum(contrib)

    @pl.when(rb == 0)
    def _():
        out_ref[0, 0] = val

    @pl.when(rb > 0)
    def _():
        out_ref[0, 0] = out_ref[0, 0] + val


@jax.jit
def kernel(pred, target):
    B = pred.shape[0]
    nb = B // _R
    tgt3 = target.astype(jnp.int32).reshape(nb, 1, _R)

    acc = pl.pallas_call(
        _loss_kernel,
        grid=(nb,),
        in_specs=[
            pl.BlockSpec((1, 1, _R), lambda rb: (rb, 0, 0),
                         memory_space=pltpu.SMEM),
            pl.BlockSpec((1, 1, _R), lambda rb: (rb, 0, 0)),
            pl.BlockSpec((_R, _CPAD), lambda rb: (rb, 0)),
        ],
        out_specs=pl.BlockSpec(
            (1, 1), lambda rb: (0, 0), memory_space=pltpu.SMEM),
        out_shape=jax.ShapeDtypeStruct((1, 1), jnp.float32),
        scratch_shapes=[
            pltpu.VMEM((_R, _L), jnp.float32),
        ],
    )(tgt3, tgt3, pred)

    k0 = (_C - 1) * _EPS * math.log(_EPS) + _CONF * math.log(_CONF)
    return (acc[0, 0] + B * k0) / (B * _C)


# speedup vs baseline: 2.8725x; 1.0262x over previous
import math

import jax
import jax.numpy as jnp
from jax.experimental import pallas as pl
from jax.experimental.pallas import tpu as pltpu

_C = 100000
_SMOOTHING = 0.1
_CONF = 1.0 - _SMOOTHING
_EPS = _SMOOTHING / (_C - 1)

_R = 8
_L = 128
_NT = -(-_C // _L)
_CPAD = _NT * _L
_HT = _NT // 2
_HPAD = _HT * _L
_TAIL_REM = _C - (_NT - 1) * _L
_NCH = 16
_SEC = 8192
_NSEC = -(-_C // _SEC)
_NEG_INF = float("-inf")


def _tree(vals, op):
    while len(vals) > 1:
        nxt = [op(vals[i], vals[i + 1]) for i in range(0, len(vals) - 1, 2)]
        if len(vals) % 2:
            nxt.append(vals[-1])
        vals = nxt
    return vals[0]


def _tile(refs, t):
    if t < _HT:
        return refs[0][:, pl.ds(t * _L, _L)]
    return refs[1][:, pl.ds((t - _HT) * _L, _L)]


def _lane_iota():
    return jax.lax.broadcasted_iota(jnp.int32, (_R, _L), 1)


def _chains(n):
    per = -(-n // _NCH)
    for c in range(_NCH):
        lo = c * per
        hi = min(lo + per, n)
        if lo < hi:
            yield range(lo, hi)


def _masked(x, t, fill):
    if t == _NT - 1:
        return jnp.where(_lane_iota() < _TAIL_REM, x, fill)
    return x


def _sweep_max(pred_ref):
    accs = []
    for chain in _chains(_NT):
        acc = None
        for t in chain:
            x = _masked(_tile(pred_ref, t), t, _NEG_INF)
            acc = x if acc is None else jnp.maximum(acc, x)
        accs.append(acc)
    return _tree(accs, jnp.maximum)


def _sweep_stats(pred_ref, m):
    se_accs = []
    sp_accs = []
    for chain in _chains(_NT):
        se = None
        sp = None
        for t in chain:
            x = _tile(pred_ref, t)
            e = jnp.exp(_masked(x, t, _NEG_INF) - m)
            xs = _masked(x, t, 0.0)
            se = e if se is None else se + e
            sp = xs if sp is None else sp + xs
        se_accs.append(se)
        sp_accs.append(sp)
    return _tree(se_accs, jnp.add), _tree(sp_accs, jnp.add)


def _loss_kernel(tgt_s_ref, tgt_v_ref, pred_lo_ref, pred_hi_ref, out_ref, ts_ref):
    pred_ref = (pred_lo_ref, pred_hi_ref)
    rb = pl.program_id(0)

    ts_ref[...] = jnp.zeros((_R, _L), jnp.float32)

    tgtv = jnp.broadcast_to(tgt_v_ref[0, 0, :].reshape(_R, 1), (_R, _L))
    li = _lane_iota()
    for sec in range(_NSEC):
        lo = sec * _SEC
        hi = min(lo + _SEC, _C)
        hit = None
        for i in range(_R):
            t = tgt_s_ref[0, 0, i]
            h = jnp.logical_and(t >= lo, t < hi)
            hit = h if hit is None else jnp.logical_or(hit, h)

        @pl.when(hit)
        def _tsel(lo=lo, hi=hi):
            accs = []
            t0 = lo // _L
            t1 = -(-hi // _L)
            for g0 in range(t0, t1, 8):
                acc = None
                for t in range(g0, min(g0 + 8, t1)):
                    col = li + t * _L
                    v = jnp.where(col == tgtv, _tile(pred_ref, t), 0.0)
                    acc = v if acc is None else acc + v
                accs.append(acc)
            ts_ref[...] = ts_ref[...] + _tree(accs, jnp.add)

    m = _sweep_max(pred_ref)
    se, sp = _sweep_stats(pred_ref, m)

    mx = jnp.max(m, axis=1, keepdims=True)
    sx = jnp.sum(se * jnp.exp(m - mx), axis=1, keepdims=True)
    spx = jnp.sum(sp, axis=1, keepdims=True)
    tsx = jnp.sum(ts_ref[...], axis=1, keepdims=True)
    lse = mx + jnp.log(sx)
    rowsum_logp = spx - _C * lse
    logp_t = tsx - lse
    contrib = -(_EPS * rowsum_logp + (_CONF - _EPS) * logp_t)
    val = jnp.sum(contrib)

    @pl.when(rb == 0)
    def _():
        out_ref[0, 0] = val

    @pl.when(rb > 0)
    def _():
        out_ref[0, 0] = out_ref[0, 0] + val


@jax.jit
def kernel(pred, target):
    B = pred.shape[0]
    nb = B // _R
    tgt3 = target.astype(jnp.int32).reshape(nb, 1, _R)

    acc = pl.pallas_call(
        _loss_kernel,
        grid=(nb,),
        in_specs=[
            pl.BlockSpec((1, 1, _R), lambda rb: (rb, 0, 0),
                         memory_space=pltpu.SMEM),
            pl.BlockSpec((1, 1, _R), lambda rb: (rb, 0, 0)),
            pl.BlockSpec((_R, _HPAD), lambda rb: (rb, 0)),
            pl.BlockSpec((_R, _HPAD), lambda rb: (rb, 1)),
        ],
        out_specs=pl.BlockSpec(
            (1, 1), lambda rb: (0, 0), memory_space=pltpu.SMEM),
        out_shape=jax.ShapeDtypeStruct((1, 1), jnp.float32),
        scratch_shapes=[
            pltpu.VMEM((_R, _L), jnp.float32),
        ],
    )(tgt3, tgt3, pred, pred)

    k0 = (_C - 1) * _EPS * math.log(_EPS) + _CONF * math.log(_CONF)
    return (acc[0, 0] + B * k0) / (B * _C)
